# baseline (speedup 1.0000x reference)
import functools

import jax
import jax.numpy as jnp
from jax import lax
from jax.experimental import pallas as pl
from jax.experimental.pallas import tpu as pltpu
from jax.experimental.pallas import tpu_sc as plsc

N = 10000
E = 320000
D = 128
NPAD = 10240
NC, NS = 2, 16
NW = NC * NS
EPW = E // NW
CH = 80
NCH = EPW // CH
RPT = NPAD // NS
ZR = CH


def _sc_aggregate(x, edge_index, zeros):
    mesh = plsc.VectorSubcoreMesh(
        core_axis_name="core", subcore_axis_name="subcore",
        num_cores=NC, num_subcores=NS)

    @functools.partial(
        pl.kernel,
        out_type=(jax.ShapeDtypeStruct((NC, NPAD, D), jnp.float32),
                  jax.ShapeDtypeStruct((NW, N), jnp.float32)),
        mesh=mesh,
        compiler_params=pltpu.CompilerParams(
            use_tc_tiling_on_sc=False, needs_layout_passes=False),
        scratch_types=[
            pltpu.VMEM((2, CH), jnp.int32),
            pltpu.VMEM((2, CH), jnp.int32),
            pltpu.VMEM((CH, D), jnp.float32),
            pltpu.VMEM((CH, D), jnp.float32),
            pltpu.VMEM((N,), jnp.float32),
            pltpu.SemaphoreType.DMA,
            pltpu.SemaphoreType.DMA,
            pltpu.SemaphoreType.DMA,
            pltpu.SemaphoreType.DMA,
            pltpu.VMEM_SHARED((NPAD, D), jnp.float32),
        ],
    )
    def agg_kernel(x_hbm, ei_hbm, z_hbm, out_hbm, cnt_hbm,
                   ebuf0, ebuf1, rows0, rows1, hist,
                   gsem0, gsem1, isem0, isem1, acc):
        cid = lax.axis_index("core")
        sid = lax.axis_index("subcore")
        wid = cid * NS + sid
        base = wid * EPW

        @pl.loop(0, RPT, step=ZR)
        def _(r):
            pltpu.sync_copy(z_hbm, acc.at[pl.ds(sid * RPT + r, ZR)])

        @pl.loop(0, N, step=16)
        def _(i):
            hist[pl.ds(i, 16)] = jnp.zeros((16,), jnp.float32)

        plsc.subcore_barrier()

        ones16 = jnp.ones((16,), jnp.float32)

        def idx_load(g, ebuf, sem):
            pltpu.async_copy(
                ei_hbm.at[pl.ds(0, 2), pl.ds(base + g * CH, CH)], ebuf, sem)

        def idx_wait(g, ebuf, sem):
            pltpu.make_async_copy(
                ei_hbm.at[pl.ds(0, 2), pl.ds(base + g * CH, CH)],
                ebuf, sem).wait()

        def gather(ebuf, rows, sem):
            pltpu.async_copy(x_hbm.at[ebuf.at[0]], rows, sem)

        def gwait(ebuf, rows, sem):
            pltpu.make_async_copy(x_hbm.at[ebuf.at[0]], rows, sem).wait()

        def scatter(ebuf, rows):
            pltpu.sync_copy(rows, acc.at[ebuf.at[1]], add=True)

        def count(ebuf):
            @pl.loop(0, CH, step=16)
            def _(k):
                dvec = ebuf[1, pl.ds(k, 16)]
                plsc.addupdate_scatter(hist, [dvec], ones16)

        idx_load(0, ebuf0, isem0)
        idx_wait(0, ebuf0, isem0)
        gather(ebuf0, rows0, gsem0)
        idx_load(1, ebuf1, isem1)

        @pl.loop(0, NCH - 1, step=2)
        def _(g):
            gwait(ebuf0, rows0, gsem0)
            idx_wait(g + 1, ebuf1, isem1)
            gather(ebuf1, rows1, gsem1)
            scatter(ebuf0, rows0)
            count(ebuf0)

            @pl.when(g + 2 < NCH)
            def _():
                idx_load(g + 2, ebuf0, isem0)

            gwait(ebuf1, rows1, gsem1)

            @pl.when(g + 2 < NCH)
            def _():
                idx_wait(g + 2, ebuf0, isem0)
                gather(ebuf0, rows0, gsem0)

            scatter(ebuf1, rows1)
            count(ebuf1)

            @pl.when(g + 3 < NCH)
            def _():
                idx_load(g + 3, ebuf1, isem1)

        gwait(ebuf0, rows0, gsem0)
        scatter(ebuf0, rows0)
        count(ebuf0)

        plsc.subcore_barrier()
        pltpu.sync_copy(acc.at[pl.ds(sid * RPT, RPT)],
                        out_hbm.at[cid, pl.ds(sid * RPT, RPT)])
        pltpu.sync_copy(hist, cnt_hbm.at[wid])

    return agg_kernel(x, edge_index, zeros)


def _tc_right(x, W_r, b_l):
    BR = 2000

    def body(x_ref, wr_ref, b_ref, o_ref):
        o_ref[...] = (
            jnp.dot(x_ref[...], wr_ref[...], preferred_element_type=jnp.float32)
            + b_ref[...]
        )

    return pl.pallas_call(
        body,
        grid=(N // BR,),
        in_specs=[
            pl.BlockSpec((BR, D), lambda i: (i, 0)),
            pl.BlockSpec((D, D), lambda i: (0, 0)),
            pl.BlockSpec((1, D), lambda i: (0, 0)),
        ],
        out_specs=pl.BlockSpec((BR, D), lambda i: (i, 0)),
        out_shape=jax.ShapeDtypeStruct((N, D), jnp.float32),
    )(x, W_r, b_l.reshape(1, D))


def _tc_combine(partials, counts, z, W_l):
    BR = 2000

    def body(p_ref, c_ref, z_ref, wl_ref, o_ref):
        agg = p_ref[0] + p_ref[1]
        cnt = jnp.sum(c_ref[...], axis=1)[:, None]
        mean = agg / jnp.maximum(cnt, 1.0)
        o_ref[...] = (
            jnp.dot(mean, wl_ref[...], preferred_element_type=jnp.float32)
            + z_ref[...]
        )

    return pl.pallas_call(
        body,
        grid=(N // BR,),
        in_specs=[
            pl.BlockSpec((NC, BR, D), lambda i: (0, i, 0)),
            pl.BlockSpec((BR, NW), lambda i: (i, 0)),
            pl.BlockSpec((BR, D), lambda i: (i, 0)),
            pl.BlockSpec((D, D), lambda i: (0, 0)),
        ],
        out_specs=pl.BlockSpec((BR, D), lambda i: (i, 0)),
        out_shape=jax.ShapeDtypeStruct((N, D), jnp.float32),
    )(partials, counts, z, W_l)


def kernel(x, edge_index, W_l, b_l, W_r):
    zeros = jnp.zeros((ZR, D), jnp.float32)
    partials, counts = _sc_aggregate(x, edge_index, zeros)
    z = _tc_right(x, W_r, b_l)
    return _tc_combine(partials, counts.T, z, W_l)
